# baseline (speedup 1.0000x reference)
import functools

import jax
import jax.numpy as jnp
from jax import lax
from jax.experimental import pallas as pl
from jax.experimental.pallas import tpu as pltpu
from jax.experimental.pallas import tpu_sc as plsc

HIDDEN = 1024
NUM_EXPERTS = 8
TOP_K = 2
EPS = 1e-06
SCALAR_ROOT = HIDDEN ** (-0.5)

TOKENS = 32768
TC_BLOCK = 4096

NC = 2
NS = 16
L = 16
NW = NC * NS
CHUNK = TOKENS // NW


def _router_block(h_ref, s_ref, wpt_ref, pt_ref):
    h = h_ref[...]
    var = jnp.sum(h * h, axis=1, keepdims=True) * (1.0 / HIDDEN)
    r = lax.rsqrt(var + EPS)
    hh = ((h * r) * s_ref[...]) * SCALAR_ROOT
    st = lax.dot_general(
        wpt_ref[...].astype(jnp.bfloat16), hh.astype(jnp.bfloat16),
        (((1,), (1,)), ((), ())),
        preferred_element_type=jnp.float32,
    )
    m = jnp.max(st, axis=0, keepdims=True)
    e = jnp.exp(st - m)
    pt_ref[...] = e / jnp.sum(e, axis=0, keepdims=True)


def _router_probs(hidden_states, scale, wpt):
    return pl.pallas_call(
        _router_block,
        grid=(TOKENS // TC_BLOCK,),
        in_specs=[
            pl.BlockSpec((TC_BLOCK, HIDDEN), lambda i: (i, 0)),
            pl.BlockSpec((1, HIDDEN), lambda i: (0, 0)),
            pl.BlockSpec((NUM_EXPERTS, HIDDEN), lambda i: (0, 0)),
        ],
        out_specs=pl.BlockSpec((NUM_EXPERTS, TC_BLOCK), lambda i: (0, i)),
        out_shape=jax.ShapeDtypeStruct((NUM_EXPERTS, TOKENS), jnp.float32),
        name="router_tc",
    )(hidden_states, scale.reshape(1, HIDDEN), wpt)


def _topk_body(pt_hbm, pes_hbm, w1_hbm, w2_hbm, i1_hbm, i2_hbm,
               p_v, pes_v, w1_v, w2_v, i1_v, i2_v, sem):
    wid = lax.axis_index("s") * NC + lax.axis_index("c")
    base = wid * CHUNK
    pltpu.sync_copy(pes_hbm, pes_v)
    copies = [
        pltpu.async_copy(pt_hbm.at[e, pl.ds(base, CHUNK)], p_v.at[e], sem)
        for e in range(NUM_EXPERTS)
    ]
    for c in copies:
        c.wait()

    UNROLL = 8

    def body(j, _):
        for u in range(UNROLL):
            sl = pl.ds((j * UNROLL + u) * L, L)
            p = [p_v[e, sl] for e in range(NUM_EXPERTS)]
            m1 = p[0]
            i1 = jnp.zeros((L,), jnp.int32)
            for e in range(1, NUM_EXPERTS):
                c = p[e] > m1
                m1 = jnp.where(c, p[e], m1)
                i1 = jnp.where(c, e, i1)
            m2 = jnp.full((L,), -jnp.inf, jnp.float32)
            i2 = jnp.zeros((L,), jnp.int32)
            for e in range(NUM_EXPERTS):
                c = (p[e] > m2) & (i1 != e)
                m2 = jnp.where(c, p[e], m2)
                i2 = jnp.where(c, e, i2)
            inv = 1.0 / (m1 + m2)
            w1_v[sl] = m1 * inv * plsc.load_gather(pes_v, [i1])
            w2_v[sl] = m2 * inv * plsc.load_gather(pes_v, [i2])
            i1_v[sl] = i1
            i2_v[sl] = i2
        return 0

    lax.fori_loop(0, CHUNK // (L * UNROLL), body, 0)
    pltpu.sync_copy(w1_v, w1_hbm.at[pl.ds(base, CHUNK)])
    pltpu.sync_copy(w2_v, w2_hbm.at[pl.ds(base, CHUNK)])
    pltpu.sync_copy(i1_v, i1_hbm.at[pl.ds(base, CHUNK)])
    pltpu.sync_copy(i2_v, i2_hbm.at[pl.ds(base, CHUNK)])


def _topk_sc(pt, pes_pad):
    mesh = plsc.VectorSubcoreMesh(core_axis_name="c", subcore_axis_name="s")
    fn = functools.partial(
        pl.kernel,
        out_type=(
            jax.ShapeDtypeStruct((TOKENS,), jnp.float32),
            jax.ShapeDtypeStruct((TOKENS,), jnp.float32),
            jax.ShapeDtypeStruct((TOKENS,), jnp.int32),
            jax.ShapeDtypeStruct((TOKENS,), jnp.int32),
        ),
        mesh=mesh,
        scratch_types=[
            pltpu.VMEM((NUM_EXPERTS, CHUNK), jnp.float32),
            pltpu.VMEM((L,), jnp.float32),
            pltpu.VMEM((CHUNK,), jnp.float32),
            pltpu.VMEM((CHUNK,), jnp.float32),
            pltpu.VMEM((CHUNK,), jnp.int32),
            pltpu.VMEM((CHUNK,), jnp.int32),
            pltpu.SemaphoreType.DMA,
        ],
        compiler_params=pltpu.CompilerParams(
            needs_layout_passes=False, use_tc_tiling_on_sc=False
        ),
    )(_topk_body)
    return fn(pt, pes_pad)


def kernel(hidden_states, scale, per_expert_scale, W_proj):
    pt = _router_probs(hidden_states, scale, W_proj)
    pes_pad = jnp.pad(per_expert_scale, (0, L - NUM_EXPERTS))
    w1, w2, i1, i2 = _topk_sc(pt, pes_pad)
    probs = pt.T
    w = jnp.stack([w1, w2], axis=1)
    i = jnp.stack([i1, i2], axis=1)
    return probs, w, i


# speedup vs baseline: 1.0032x; 1.0032x over previous
import functools

import jax
import jax.numpy as jnp
from jax import lax
from jax.experimental import pallas as pl
from jax.experimental.pallas import tpu as pltpu
from jax.experimental.pallas import tpu_sc as plsc

HIDDEN = 1024
NUM_EXPERTS = 8
TOP_K = 2
EPS = 1e-06
SCALAR_ROOT = HIDDEN ** (-0.5)

TOKENS = 32768
TC_BLOCK = 4096

NC = 2
NS = 16
L = 16
NW = NC * NS
CHUNK = TOKENS // NW


def _router_block(h_ref, s_ref, wpt_ref, pt_ref):
    h = h_ref[...]
    var = jnp.sum(h * h, axis=1, keepdims=True) * (1.0 / HIDDEN)
    r = lax.rsqrt(var + EPS)
    hh = ((h * r) * s_ref[...]) * SCALAR_ROOT
    st = lax.dot_general(
        wpt_ref[...].astype(jnp.bfloat16), hh.astype(jnp.bfloat16),
        (((1,), (1,)), ((), ())),
        preferred_element_type=jnp.float32,
    )
    m = jnp.max(st, axis=0, keepdims=True)
    e = jnp.exp(st - m)
    pt_ref[...] = e / jnp.sum(e, axis=0, keepdims=True)


def _router_probs(hidden_states, scale, wpt):
    return pl.pallas_call(
        _router_block,
        grid=(TOKENS // TC_BLOCK,),
        in_specs=[
            pl.BlockSpec((TC_BLOCK, HIDDEN), lambda i: (i, 0)),
            pl.BlockSpec((1, HIDDEN), lambda i: (0, 0)),
            pl.BlockSpec((NUM_EXPERTS, HIDDEN), lambda i: (0, 0)),
        ],
        out_specs=pl.BlockSpec((NUM_EXPERTS, TC_BLOCK), lambda i: (0, i)),
        out_shape=jax.ShapeDtypeStruct((NUM_EXPERTS, TOKENS), jnp.float32),
        name="router_tc",
    )(hidden_states, scale.reshape(1, HIDDEN), wpt)


def _topk_body(pt_hbm, pes_hbm, w1_hbm, w2_hbm, i1_hbm, i2_hbm,
               p_v, pes_v, w1_v, w2_v, i1_v, i2_v, sem):
    wid = lax.axis_index("s") * NC + lax.axis_index("c")
    base = wid * CHUNK
    pltpu.sync_copy(pes_hbm, pes_v)
    copies = [
        pltpu.async_copy(pt_hbm.at[e, pl.ds(base, CHUNK)], p_v.at[e], sem)
        for e in range(NUM_EXPERTS)
    ]
    for c in copies:
        c.wait()

    UNROLL = 8

    def body(j, _):
        for u in range(UNROLL):
            sl = pl.ds((j * UNROLL + u) * L, L)
            p = [p_v[e, sl] for e in range(NUM_EXPERTS)]
            m1 = p[0]
            i1 = jnp.zeros((L,), jnp.int32)
            for e in range(1, NUM_EXPERTS):
                c = p[e] > m1
                m1 = jnp.where(c, p[e], m1)
                i1 = jnp.where(c, e, i1)
            m2 = jnp.full((L,), -jnp.inf, jnp.float32)
            i2 = jnp.zeros((L,), jnp.int32)
            for e in range(NUM_EXPERTS):
                c = (p[e] > m2) & (i1 != e)
                m2 = jnp.where(c, p[e], m2)
                i2 = jnp.where(c, e, i2)
            inv = 1.0 / (m1 + m2)
            w1_v[sl] = m1 * inv * plsc.load_gather(pes_v, [i1])
            w2_v[sl] = m2 * inv * plsc.load_gather(pes_v, [i2])
            i1_v[sl] = i1
            i2_v[sl] = i2
        return 0

    lax.fori_loop(0, CHUNK // (L * UNROLL), body, 0)
    pltpu.sync_copy(w1_v, w1_hbm.at[pl.ds(base, CHUNK)])
    pltpu.sync_copy(w2_v, w2_hbm.at[pl.ds(base, CHUNK)])
    pltpu.sync_copy(i1_v, i1_hbm.at[pl.ds(base, CHUNK)])
    pltpu.sync_copy(i2_v, i2_hbm.at[pl.ds(base, CHUNK)])


def _topk_sc(pt, pes_pad):
    mesh = plsc.VectorSubcoreMesh(core_axis_name="c", subcore_axis_name="s")
    fn = functools.partial(
        pl.kernel,
        out_type=(
            jax.ShapeDtypeStruct((TOKENS,), jnp.float32),
            jax.ShapeDtypeStruct((TOKENS,), jnp.float32),
            jax.ShapeDtypeStruct((TOKENS,), jnp.int32),
            jax.ShapeDtypeStruct((TOKENS,), jnp.int32),
        ),
        mesh=mesh,
        scratch_types=[
            pltpu.VMEM((NUM_EXPERTS, CHUNK), jnp.float32),
            pltpu.VMEM((L,), jnp.float32),
            pltpu.VMEM((CHUNK,), jnp.float32),
            pltpu.VMEM((CHUNK,), jnp.float32),
            pltpu.VMEM((CHUNK,), jnp.int32),
            pltpu.VMEM((CHUNK,), jnp.int32),
            pltpu.SemaphoreType.DMA,
        ],
        compiler_params=pltpu.CompilerParams(
            needs_layout_passes=False, use_tc_tiling_on_sc=False
        ),
    )(_topk_body)
    return fn(pt, pes_pad)


def kernel(hidden_states, scale, per_expert_scale, W_proj):
    pt = _router_probs(hidden_states, scale, W_proj)
    probs = pt.T
    pes_pad = jnp.pad(per_expert_scale, (0, L - NUM_EXPERTS))
    w1, w2, i1, i2 = _topk_sc(pt, pes_pad)
    w = jnp.stack([w1, w2], axis=1)
    i = jnp.stack([i1, i2], axis=1)
    return probs, w, i
